# baseline (speedup 1.0000x reference)
import functools

import jax
import jax.numpy as jnp
from jax import lax
from jax.experimental import pallas as pl
from jax.experimental.pallas import tpu as pltpu
from jax.experimental.pallas import tpu_sc as plsc

_INFO = plsc.get_sparse_core_info()
_NC = _INFO.num_cores
_NS = _INFO.num_subcores
_LANES = _INFO.num_lanes
_NW = _NC * _NS

_R = 16
_CP = 256
_STAGE = 8192


def _gather_mul(x, idx, weight):
    b, l = x.shape
    w_len = weight.shape[0]
    seg = w_len // _NS
    rows_per_worker = b // _NW
    n_chunks = rows_per_worker // _R
    mesh = plsc.VectorSubcoreMesh(core_axis_name="c", subcore_axis_name="s")

    @functools.partial(
        pl.kernel,
        mesh=mesh,
        out_type=jax.ShapeDtypeStruct((b, l), jnp.float32),
        scratch_types=[
            pltpu.VMEM_SHARED((w_len,), jnp.float32),
            pltpu.VMEM((_STAGE,), jnp.float32),
        ] + [pltpu.VMEM((_R, _CP), jnp.int32)] * 4
          + [pltpu.VMEM((_R, _CP), jnp.float32)] * 8
          + [pltpu.SemaphoreType.DMA] * 16,
        compiler_params=pltpu.CompilerParams(use_tc_tiling_on_sc=True),
    )
    def k(x_hbm, idx_hbm, w_hbm, out_hbm, w_sh, stg_v, idx_v0, idx_v1,
          idx_v2, idx_v3, w_v0, w_v1, w_v2, w_v3, x_v0, x_v1, x_v2, x_v3,
          si0, si1, si2, si3, sx0, sx1, sx2, sx3,
          sg0, sg1, sg2, sg3, so0, so1, so2, so3):
        idx_v = (idx_v0, idx_v1, idx_v2, idx_v3)
        w_v = (w_v0, w_v1, w_v2, w_v3)
        x_v = (x_v0, x_v1, x_v2, x_v3)
        sem_i = (si0, si1, si2, si3)
        sem_x = (sx0, sx1, sx2, sx3)
        sem_g = (sg0, sg1, sg2, sg3)
        sem_o = (so0, so1, so2, so3)
        sid = lax.axis_index("s")
        wid = sid * _NC + lax.axis_index("c")
        base = wid * rows_per_worker

        def rows(c):
            return pl.ds(base + c * _R, _R)

        h_i, h_x, h_o = {}, {}, {}

        def stage(c):
            bb = c % 4
            h_i[c] = pltpu.async_copy(idx_hbm.at[rows(c), :], idx_v[bb],
                                      sem_i[bb])
            h_x[c] = pltpu.async_copy(x_hbm.at[rows(c), :], x_v[bb],
                                      sem_x[bb])

        for c0 in range(min(4, n_chunks)):
            stage(c0)
        for p in range(seg // _STAGE):
            sl = pl.ds(sid * seg + p * _STAGE, _STAGE)
            pltpu.sync_copy(w_hbm.at[sl], stg_v)
            pltpu.sync_copy(stg_v, w_sh.at[sl])
        plsc.subcore_barrier()

        def gather_streams(bb, fn):
            def body(r, _):
                for j in range(_CP // 128):
                    s = pl.ds(j * 128, 128)
                    fn(pltpu.make_async_copy(
                        w_sh.at[idx_v[bb].at[r, s]], w_v[bb].at[r, s],
                        sem_g[bb]))
                return 0

            lax.fori_loop(0, _R, body, 0)

        def fire_gather(c):
            bb = c % 4
            h_i[c].wait()
            if c >= 4:
                h_o[c - 4].wait()
            gather_streams(bb, lambda cp: cp.start())

        fire_gather(0)

        for c in range(n_chunks):
            bb = c % 4
            if c + 1 < n_chunks:
                fire_gather(c + 1)

            gather_streams(bb, lambda cp: cp.wait())
            h_x[c].wait()

            wb, xb = w_v[bb], x_v[bb]

            def body(r, _):
                for j in range(_CP // _LANES):
                    s = pl.ds(j * _LANES, _LANES)
                    wb[r, s] = wb[r, s] * xb[r, s]
                return 0

            lax.fori_loop(0, _R, body, 0)

            h_o[c] = pltpu.async_copy(w_v[bb], out_hbm.at[rows(c), :],
                                      sem_o[bb])
            if c + 4 < n_chunks:
                stage(c + 4)

        for t in range(min(4, n_chunks)):
            h_o[n_chunks - 1 - t].wait()

    return k(x, idx, weight)


def kernel(x, index, weight):
    shape = x.shape
    n = x.size
    rows = n // _CP
    x2 = x.astype(jnp.float32).reshape(rows, _CP)
    idx2 = index.astype(jnp.int32).reshape(rows, _CP)

    w_flat = weight.reshape(weight.size).astype(jnp.float32)
    wpad = (-w_flat.size) % (_NS * _STAGE)
    if wpad:
        w_flat = jnp.pad(w_flat, (0, wpad))

    return _gather_mul(x2, idx2, w_flat).reshape(shape)


# speedup vs baseline: 1.0033x; 1.0033x over previous
import functools

import jax
import jax.numpy as jnp
from jax import lax
from jax.experimental import pallas as pl
from jax.experimental.pallas import tpu as pltpu
from jax.experimental.pallas import tpu_sc as plsc

_INFO = plsc.get_sparse_core_info()
_NC = _INFO.num_cores
_NS = _INFO.num_subcores
_LANES = _INFO.num_lanes
_NW = _NC * _NS

_R = 16
_CP = 256
_STAGE = 8192


def _gather_mul(x, idx, weight):
    b, l = x.shape
    w_len = weight.shape[0]
    seg = w_len // _NS
    rows_per_worker = b // _NW
    n_chunks = rows_per_worker // _R
    mesh = plsc.VectorSubcoreMesh(core_axis_name="c", subcore_axis_name="s")

    @functools.partial(
        pl.kernel,
        mesh=mesh,
        out_type=jax.ShapeDtypeStruct((b, l), jnp.float32),
        scratch_types=[
            pltpu.VMEM_SHARED((w_len,), jnp.float32),
            pltpu.VMEM((_STAGE,), jnp.float32),
        ] + [pltpu.VMEM((_R, _CP), jnp.int32)] * 3
          + [pltpu.VMEM((_R, _CP), jnp.float32)] * 6
          + [pltpu.SemaphoreType.DMA] * 12,
        compiler_params=pltpu.CompilerParams(use_tc_tiling_on_sc=True),
    )
    def k(x_hbm, idx_hbm, w_hbm, out_hbm, w_sh, stg_v, idx_v0, idx_v1,
          idx_v2, w_v0, w_v1, w_v2, x_v0, x_v1, x_v2,
          si0, si1, si2, sx0, sx1, sx2, sg0, sg1, sg2, so0, so1, so2):
        idx_v = (idx_v0, idx_v1, idx_v2)
        w_v = (w_v0, w_v1, w_v2)
        x_v = (x_v0, x_v1, x_v2)
        sem_i = (si0, si1, si2)
        sem_x = (sx0, sx1, sx2)
        sem_g = (sg0, sg1, sg2)
        sem_o = (so0, so1, so2)
        sid = lax.axis_index("s")
        wid = sid * _NC + lax.axis_index("c")
        base = wid * rows_per_worker

        def rows(c):
            return pl.ds(base + c * _R, _R)

        h_i, h_x, h_o = {}, {}, {}

        def stage(c):
            bb = c % 3
            h_i[c] = pltpu.async_copy(idx_hbm.at[rows(c), :], idx_v[bb],
                                      sem_i[bb])
            h_x[c] = pltpu.async_copy(x_hbm.at[rows(c), :], x_v[bb],
                                      sem_x[bb])

        for c0 in range(min(3, n_chunks)):
            stage(c0)
        for p in range(seg // _STAGE):
            sl = pl.ds(sid * seg + p * _STAGE, _STAGE)
            pltpu.sync_copy(w_hbm.at[sl], stg_v)
            pltpu.sync_copy(stg_v, w_sh.at[sl])
        plsc.subcore_barrier()

        def gather_streams(bb, fn):
            def body(r, _):
                for j in range(_CP // 128):
                    s = pl.ds(j * 128, 128)
                    fn(pltpu.make_async_copy(
                        w_sh.at[idx_v[bb].at[r, s]], w_v[bb].at[r, s],
                        sem_g[bb]))
                return 0

            lax.fori_loop(0, _R, body, 0)

        def fire_gather(c):
            bb = c % 3
            h_i[c].wait()
            if c >= 3:
                h_o[c - 3].wait()
            gather_streams(bb, lambda cp: cp.start())

        fire_gather(0)

        for c in range(n_chunks):
            bb = c % 3
            if c + 1 < n_chunks:
                fire_gather(c + 1)

            gather_streams(bb, lambda cp: cp.wait())
            h_x[c].wait()

            wb, xb = w_v[bb], x_v[bb]

            def body(r, _):
                for j in range(_CP // _LANES):
                    s = pl.ds(j * _LANES, _LANES)
                    wb[r, s] = wb[r, s] * xb[r, s]
                return 0

            lax.fori_loop(0, _R, body, 0)

            h_o[c] = pltpu.async_copy(w_v[bb], out_hbm.at[rows(c), :],
                                      sem_o[bb])
            if c + 3 < n_chunks:
                stage(c + 3)

        for t in range(min(3, n_chunks)):
            h_o[n_chunks - 1 - t].wait()

    return k(x, idx, weight)


def kernel(x, index, weight):
    shape = x.shape
    n = x.size
    rows = n // _CP
    x2 = x.astype(jnp.float32).reshape(rows, _CP)
    idx2 = index.astype(jnp.int32).reshape(rows, _CP)

    w_flat = weight.reshape(weight.size).astype(jnp.float32)
    wpad = (-w_flat.size) % (_NS * _STAGE)
    if wpad:
        w_flat = jnp.pad(w_flat, (0, wpad))

    return _gather_mul(x2, idx2, w_flat).reshape(shape)
